# Initial kernel scaffold; baseline (speedup 1.0000x reference)
import jax
import jax.numpy as jnp
from jax.experimental import pallas as pl


def kernel(sim_mat, nrows, ncols):
    raise NotImplementedError("write your pallas kernel here")


# speedup vs baseline: 1.8522x; 1.8522x over previous
import jax
import jax.numpy as jnp
from jax.experimental import pallas as pl
from jax.experimental.pallas import tpu as pltpu

_BR = 256


def _softmax_block(nrows_ref, ncols_ref, x_ref, o_ref):
    b = pl.program_id(0)
    rb = pl.program_id(1)
    nr = nrows_ref[b]
    nc = ncols_ref[b]
    br, c = o_ref.shape[1], o_ref.shape[2]
    row0 = rb * br

    @pl.when(row0 >= nr)
    def _():
        o_ref[...] = jnp.zeros_like(o_ref)

    @pl.when(row0 < nr)
    def _():
        x = x_ref[0]
        rows = row0 + jax.lax.broadcasted_iota(jnp.int32, (br, c), 0)
        cols = jax.lax.broadcasted_iota(jnp.int32, (br, c), 1)
        mask = (rows < nr) & (cols < nc)
        masked = jnp.where(mask, x, -jnp.inf)
        m = jnp.max(masked, axis=1, keepdims=True)
        safe_m = jnp.where(jnp.isfinite(m), m, 0.0)
        e = jnp.where(mask, jnp.exp(x - safe_m), 0.0)
        denom = jnp.sum(e, axis=1, keepdims=True)
        inv = jnp.where(denom > 0, 1.0 / jnp.maximum(denom, 1e-30), 0.0)
        o_ref[0] = e * inv


def _x_index(b, rb, nrows_ref, ncols_ref):
    nr = nrows_ref[b]
    last_active = jnp.maximum((nr + _BR - 1) // _BR - 1, 0)
    return (b, jnp.minimum(rb, last_active), 0)


def _o_index(b, rb, nrows_ref, ncols_ref):
    return (b, rb, 0)


def kernel(sim_mat, nrows, ncols):
    bsz, r, c = sim_mat.shape
    grid_spec = pltpu.PrefetchScalarGridSpec(
        num_scalar_prefetch=2,
        grid=(bsz, r // _BR),
        in_specs=[pl.BlockSpec((1, _BR, c), _x_index)],
        out_specs=pl.BlockSpec((1, _BR, c), _o_index),
    )
    return pl.pallas_call(
        _softmax_block,
        grid_spec=grid_spec,
        out_shape=jax.ShapeDtypeStruct((bsz, r, c), sim_mat.dtype),
    )(nrows.astype(jnp.int32), ncols.astype(jnp.int32), sim_mat)
